# Initial kernel scaffold; baseline (speedup 1.0000x reference)
import jax
import jax.numpy as jnp
from jax.experimental import pallas as pl


def kernel(weights, positions):
    raise NotImplementedError("write your pallas kernel here")


# speedup vs baseline: 5.8192x; 5.8192x over previous
import functools

import jax
import jax.numpy as jnp
from jax import lax
from jax.experimental import pallas as pl
from jax.experimental.pallas import tpu as pltpu
from jax.experimental.pallas import tpu_sc as plsc

EMBED_DIM = 128
NUM_CORES = 2
NUM_SUBCORES = 16
NUM_WORKERS = NUM_CORES * NUM_SUBCORES
CHUNK = 256


def _make_lookup(batch):
    assert batch % (8 * NUM_WORKERS) == 0
    b_per_w = batch // NUM_WORKERS
    assert b_per_w % CHUNK == 0
    n_chunks = b_per_w // CHUNK
    mesh = plsc.VectorSubcoreMesh(core_axis_name="c", subcore_axis_name="s")

    @functools.partial(
        pl.kernel,
        mesh=mesh,
        out_type=jax.ShapeDtypeStruct((batch, EMBED_DIM), jnp.float32),
        scratch_types=[
            pltpu.VMEM((CHUNK,), jnp.int32),
            pltpu.VMEM((CHUNK, EMBED_DIM), jnp.float32),
            pltpu.SemaphoreType.DMA,
        ],
    )
    def lookup(table_hbm, idx_hbm, out_hbm, idx_v, rows_v, sem):
        wid = lax.axis_index("s") * NUM_CORES + lax.axis_index("c")
        base = wid * b_per_w

        def body(i, carry):
            off = base + i * CHUNK
            pltpu.sync_copy(idx_hbm.at[pl.ds(off, CHUNK)], idx_v)
            pltpu.async_copy(table_hbm.at[idx_v], rows_v, sem).wait()
            pltpu.sync_copy(rows_v, out_hbm.at[pl.ds(off, CHUNK)])
            return carry

        lax.fori_loop(0, n_chunks, body, 0)

    return lookup


def kernel(weights, positions):
    flat = positions.reshape(-1).astype(jnp.int32)
    out = _make_lookup(flat.shape[0])(weights, flat)
    return out.reshape(*positions.shape, EMBED_DIM)


# speedup vs baseline: 13.0589x; 2.2441x over previous
import functools

import jax
import jax.numpy as jnp
from jax import lax
from jax.experimental import pallas as pl
from jax.experimental.pallas import tpu as pltpu
from jax.experimental.pallas import tpu_sc as plsc

EMBED_DIM = 128
NUM_TABLE_ROWS = 1024
NUM_CORES = 2
NUM_SUBCORES = 16
NUM_WORKERS = NUM_CORES * NUM_SUBCORES
CHUNK = 256


def _make_lookup(batch):
    assert batch % (8 * NUM_WORKERS) == 0
    b_per_w = batch // NUM_WORKERS
    assert b_per_w % (2 * CHUNK) == 0
    pair_steps = b_per_w // (2 * CHUNK)
    mesh = plsc.VectorSubcoreMesh(core_axis_name="c", subcore_axis_name="s")

    @functools.partial(
        pl.kernel,
        mesh=mesh,
        out_type=jax.ShapeDtypeStruct((batch, EMBED_DIM), jnp.float32),
        scratch_types=[
            pltpu.VMEM_SHARED((NUM_TABLE_ROWS, EMBED_DIM), jnp.float32),
            pltpu.VMEM((CHUNK,), jnp.int32),
            pltpu.VMEM((CHUNK,), jnp.int32),
            pltpu.VMEM((CHUNK, EMBED_DIM), jnp.float32),
            pltpu.VMEM((CHUNK, EMBED_DIM), jnp.float32),
            pltpu.SemaphoreType.DMA,
            pltpu.SemaphoreType.DMA,
        ],
    )
    def lookup(table_hbm, idx_hbm, out_hbm, tab_sh, idx0, idx1, rows0, rows1,
               sem0, sem1):
        sid = lax.axis_index("s")
        wid = sid * NUM_CORES + lax.axis_index("c")
        base = wid * b_per_w

        @pl.when(sid == 0)
        def _():
            pltpu.sync_copy(table_hbm, tab_sh)

        plsc.subcore_barrier()

        pltpu.sync_copy(idx_hbm.at[pl.ds(base, CHUNK)], idx0)
        pltpu.async_copy(tab_sh.at[idx0], rows0, sem0)

        def body(g, carry):
            off = base + 2 * g * CHUNK

            pltpu.sync_copy(idx_hbm.at[pl.ds(off + CHUNK, CHUNK)], idx1)
            pltpu.async_copy(tab_sh.at[idx1], rows1, sem1)

            pltpu.make_async_copy(tab_sh.at[idx0], rows0, sem0).wait()
            pltpu.sync_copy(rows0, out_hbm.at[pl.ds(off, CHUNK)])

            @pl.when(g + 1 < pair_steps)
            def _():
                pltpu.sync_copy(idx_hbm.at[pl.ds(off + 2 * CHUNK, CHUNK)], idx0)
                pltpu.async_copy(tab_sh.at[idx0], rows0, sem0)

            pltpu.make_async_copy(tab_sh.at[idx1], rows1, sem1).wait()
            pltpu.sync_copy(rows1, out_hbm.at[pl.ds(off + CHUNK, CHUNK)])
            return carry

        lax.fori_loop(0, pair_steps, body, 0)

    return lookup


def kernel(weights, positions):
    flat = positions.reshape(-1).astype(jnp.int32)
    out = _make_lookup(flat.shape[0])(weights, flat)
    return out.reshape(*positions.shape, EMBED_DIM)


# speedup vs baseline: 15.7756x; 1.2080x over previous
import functools

import jax
import jax.numpy as jnp
from jax import lax
from jax.experimental import pallas as pl
from jax.experimental.pallas import tpu as pltpu
from jax.experimental.pallas import tpu_sc as plsc

EMBED_DIM = 128
NUM_TABLE_ROWS = 1024
NUM_CORES = 2
NUM_SUBCORES = 16
NUM_WORKERS = NUM_CORES * NUM_SUBCORES
CHUNK = 256


def _make_lookup(batch):
    assert batch % (8 * NUM_WORKERS) == 0
    b_per_w = batch // NUM_WORKERS
    assert b_per_w % (2 * CHUNK) == 0
    n_chunks = b_per_w // CHUNK
    pair_steps = n_chunks // 2
    mesh = plsc.VectorSubcoreMesh(core_axis_name="c", subcore_axis_name="s")

    @functools.partial(
        pl.kernel,
        mesh=mesh,
        out_type=jax.ShapeDtypeStruct((batch, EMBED_DIM), jnp.float32),
        scratch_types=[
            pltpu.VMEM_SHARED((NUM_TABLE_ROWS, EMBED_DIM), jnp.float32),
            pltpu.VMEM((CHUNK,), jnp.int32),
            pltpu.VMEM((CHUNK,), jnp.int32),
            pltpu.VMEM((CHUNK, EMBED_DIM), jnp.float32),
            pltpu.VMEM((CHUNK, EMBED_DIM), jnp.float32),
            pltpu.SemaphoreType.DMA,
            pltpu.SemaphoreType.DMA,
            pltpu.SemaphoreType.DMA,
            pltpu.SemaphoreType.DMA,
        ],
    )
    def lookup(table_hbm, idx_hbm, out_hbm, tab_sh, idx0, idx1, rows0, rows1,
               gsem0, gsem1, isem0, isem1):
        sid = lax.axis_index("s")
        wid = sid * NUM_CORES + lax.axis_index("c")
        base = wid * b_per_w

        def idx_src(c):
            return idx_hbm.at[pl.ds(base + c * CHUNK, CHUNK)]

        def out_dst(c):
            return out_hbm.at[pl.ds(base + c * CHUNK, CHUNK)]

        @pl.when(sid == 0)
        def _():
            pltpu.sync_copy(table_hbm, tab_sh)

        plsc.subcore_barrier()

        pltpu.async_copy(idx_src(0), idx0, isem0)
        pltpu.async_copy(idx_src(1), idx1, isem1)
        pltpu.make_async_copy(idx_src(0), idx0, isem0).wait()
        pltpu.async_copy(tab_sh.at[idx0], rows0, gsem0)

        def body(g, carry):
            c0 = 2 * g
            more = g + 1 < pair_steps

            pltpu.make_async_copy(idx_src(c0 + 1), idx1, isem1).wait()
            pltpu.async_copy(tab_sh.at[idx1], rows1, gsem1)

            pltpu.make_async_copy(tab_sh.at[idx0], rows0, gsem0).wait()

            @pl.when(more)
            def _():
                pltpu.async_copy(idx_src(c0 + 2), idx0, isem0)

            pltpu.sync_copy(rows0, out_dst(c0))

            @pl.when(more)
            def _():
                pltpu.make_async_copy(idx_src(c0 + 2), idx0, isem0).wait()
                pltpu.async_copy(tab_sh.at[idx0], rows0, gsem0)

            pltpu.make_async_copy(tab_sh.at[idx1], rows1, gsem1).wait()

            @pl.when(more)
            def _():
                pltpu.async_copy(idx_src(c0 + 3), idx1, isem1)

            pltpu.sync_copy(rows1, out_dst(c0 + 1))
            return carry

        lax.fori_loop(0, pair_steps, body, 0)

    return lookup


def kernel(weights, positions):
    flat = positions.reshape(-1).astype(jnp.int32)
    out = _make_lookup(flat.shape[0])(weights, flat)
    return out.reshape(*positions.shape, EMBED_DIM)
